# Initial kernel scaffold; baseline (speedup 1.0000x reference)
import functools

import jax
import jax.numpy as jnp
from jax import lax
from jax.experimental import pallas as pl
from jax.experimental.pallas import tpu as pltpu
from jax.experimental.pallas import tpu_sc as plsc

_DIM = 64
_NC, _NS = 2, 16
_NW = _NC * _NS

_IDX_W = 128
_CHUNK = 512
_IDX_ROWS = _CHUNK // _IDX_W


@functools.lru_cache(maxsize=None)
def _make_gather(B, dim):
    b_per_w = B // _NW
    n_chunks = b_per_w // _CHUNK
    mesh = plsc.VectorSubcoreMesh(core_axis_name="c", subcore_axis_name="s")

    @functools.partial(
        pl.kernel,
        out_type=jax.ShapeDtypeStruct((B, dim), jnp.float32),
        mesh=mesh,
        scratch_types=[
            pltpu.VMEM((_IDX_ROWS, _IDX_W), jnp.int32),
            pltpu.VMEM((_CHUNK, dim), jnp.float32),
            pltpu.SemaphoreType.DMA,
        ],
    )
    def gather(idx_hbm, table_hbm, out_hbm, idx_v, rows_v, sem):
        wid = lax.axis_index("s") * _NC + lax.axis_index("c")
        base = wid * b_per_w

        @pl.loop(0, n_chunks)
        def _chunk(c):
            r0 = base + c * _CHUNK
            pltpu.sync_copy(idx_hbm.at[pl.ds(r0 // _IDX_W, _IDX_ROWS)], idx_v)
            copies = [
                pltpu.async_copy(
                    table_hbm.at[idx_v.at[j]],
                    rows_v.at[pl.ds(j * _IDX_W, _IDX_W)],
                    sem,
                )
                for j in range(_IDX_ROWS)
            ]
            for cp in copies:
                cp.wait()
            pltpu.sync_copy(rows_v, out_hbm.at[pl.ds(r0, _CHUNK)])

    return gather


def kernel(indices, weight):
    batch, hist = indices.shape
    B = batch * hist
    idx2d = indices.reshape(B // _IDX_W, _IDX_W)
    out = _make_gather(B, weight.shape[1])(idx2d, weight)
    return out.reshape(batch, hist, weight.shape[1])


# speedup vs baseline: 1.8464x; 1.8464x over previous
import functools

import jax
import jax.numpy as jnp
from jax import lax
from jax.experimental import pallas as pl
from jax.experimental.pallas import tpu as pltpu
from jax.experimental.pallas import tpu_sc as plsc

_DIM = 64
_NC, _NS = 2, 16
_NW = _NC * _NS

_IDX_W = 128
_CHUNK = 1024
_IDX_ROWS = _CHUNK // _IDX_W


@functools.lru_cache(maxsize=None)
def _make_gather(B, dim):
    b_per_w = B // _NW
    n_chunks = b_per_w // _CHUNK
    mesh = plsc.VectorSubcoreMesh(core_axis_name="c", subcore_axis_name="s")

    @functools.partial(
        pl.kernel,
        out_type=jax.ShapeDtypeStruct((B, dim), jnp.float32),
        mesh=mesh,
        scratch_types=[
            pltpu.VMEM((_IDX_ROWS, _IDX_W), jnp.int32),
            pltpu.VMEM((_CHUNK, dim), jnp.float32),
            pltpu.SemaphoreType.DMA,
        ],
        compiler_params=pltpu.CompilerParams(use_tc_tiling_on_sc=False),
    )
    def gather(idx_hbm, table_hbm, out_hbm, idx_v, rows_v, sem):
        wid = lax.axis_index("s") * _NC + lax.axis_index("c")
        base = wid * b_per_w

        @pl.loop(0, n_chunks)
        def _chunk(c):
            r0 = base + c * _CHUNK
            i0 = pl.multiple_of(r0 // _IDX_W, 8)
            pltpu.sync_copy(idx_hbm.at[pl.ds(i0, _IDX_ROWS)], idx_v)
            copies = [
                pltpu.async_copy(
                    table_hbm.at[idx_v.at[j]],
                    rows_v.at[pl.ds(j * _IDX_W, _IDX_W)],
                    sem,
                )
                for j in range(_IDX_ROWS)
            ]
            for cp in copies:
                cp.wait()
            pltpu.sync_copy(rows_v, out_hbm.at[pl.ds(r0, _CHUNK)])

    return gather


def kernel(indices, weight):
    batch, hist = indices.shape
    B = batch * hist
    idx2d = indices.reshape(B // _IDX_W, _IDX_W)
    out = _make_gather(B, weight.shape[1])(idx2d, weight)
    return out.reshape(batch, hist, weight.shape[1])


# speedup vs baseline: 1.8718x; 1.0137x over previous
import functools

import jax
import jax.numpy as jnp
from jax import lax
from jax.experimental import pallas as pl
from jax.experimental.pallas import tpu as pltpu
from jax.experimental.pallas import tpu_sc as plsc

_NC, _NS = 2, 16
_NW = _NC * _NS

_IDX_W = 128
_CHUNK = 512
_N_G = _CHUNK // _IDX_W
_NBUF = 3


@functools.lru_cache(maxsize=None)
def _make_gather(B, dim):
    b_per_w = B // _NW
    n_chunks = b_per_w // _CHUNK
    idx_rows_w = b_per_w // _IDX_W
    assert n_chunks >= 5
    mesh = plsc.VectorSubcoreMesh(core_axis_name="c", subcore_axis_name="s")

    @functools.partial(
        pl.kernel,
        out_type=jax.ShapeDtypeStruct((B, dim), jnp.float32),
        mesh=mesh,
        scratch_types=[
            pltpu.VMEM((idx_rows_w, _IDX_W), jnp.int32),
            [pltpu.VMEM((_CHUNK, dim), jnp.float32) for _ in range(_NBUF)],
            [pltpu.SemaphoreType.DMA for _ in range(_NBUF)],
            [pltpu.SemaphoreType.DMA for _ in range(_NBUF)],
        ],
        compiler_params=pltpu.CompilerParams(use_tc_tiling_on_sc=False),
    )
    def gather(idx_hbm, table_hbm, out_hbm, idx_v, rows, sg, sw):
        wid = lax.axis_index("s") * _NC + lax.axis_index("c")
        base = wid * b_per_w

        i0 = pl.multiple_of(wid * idx_rows_w, 8)
        pltpu.sync_copy(idx_hbm.at[pl.ds(i0, idx_rows_w)], idx_v)

        def fire_g(c, b):
            for j in range(_N_G):
                pltpu.async_copy(
                    table_hbm.at[idx_v.at[c * _N_G + j]],
                    rows[b].at[pl.ds(j * _IDX_W, _IDX_W)],
                    sg[b],
                )

        def drain_g(c, b):
            for j in range(_N_G):
                pltpu.make_async_copy(
                    table_hbm.at[idx_v.at[c * _N_G + j]],
                    rows[b].at[pl.ds(j * _IDX_W, _IDX_W)],
                    sg[b],
                ).wait()

        def fire_w(c, b):
            pltpu.async_copy(rows[b], out_hbm.at[pl.ds(base + c * _CHUNK, _CHUNK)], sw[b])

        def wait_w(b):
            pltpu.make_async_copy(rows[b], out_hbm.at[pl.ds(base, _CHUNK)], sw[b]).wait()

        def step(t, j):
            drain_g(t - 2, (j + 1) % _NBUF)
            fire_w(t - 2, (j + 1) % _NBUF)
            wait_w(j)
            fire_g(t, j)

        fire_g(0, 0)
        fire_g(1, 1)
        drain_g(0, 0)
        fire_w(0, 0)
        fire_g(2, 2)

        n_groups = (n_chunks - 3) // 3

        @pl.loop(1, 1 + n_groups)
        def _group(m):
            t = m * 3
            for j in range(_NBUF):
                step(t + j, j)

        for t in range(3 * n_groups + 3, n_chunks):
            step(t, t % _NBUF)

        for t in (n_chunks - 2, n_chunks - 1):
            drain_g(t, t % _NBUF)
            fire_w(t, t % _NBUF)
        for b in range(_NBUF):
            wait_w(b)

    return gather


def kernel(indices, weight):
    batch, hist = indices.shape
    B = batch * hist
    idx2d = indices.reshape(B // _IDX_W, _IDX_W)
    out = _make_gather(B, weight.shape[1])(idx2d, weight)
    return out.reshape(batch, hist, weight.shape[1])


# speedup vs baseline: 1.8771x; 1.0028x over previous
import functools

import jax
import jax.numpy as jnp
from jax import lax
from jax.experimental import pallas as pl
from jax.experimental.pallas import tpu as pltpu
from jax.experimental.pallas import tpu_sc as plsc

_NC, _NS = 2, 16
_NW = _NC * _NS
_CHUNK = 256
_NBUF = 4


@functools.lru_cache(maxsize=None)
def _make_gather(nrows, dim):
    rows_w = nrows // _NW
    n = rows_w // _CHUNK
    nidx = rows_w // 128
    s_per_c = _CHUNK // 128
    mesh = plsc.VectorSubcoreMesh(core_axis_name="c", subcore_axis_name="s")

    @functools.partial(
        pl.kernel,
        out_type=jax.ShapeDtypeStruct((nrows, dim), jnp.float32),
        mesh=mesh,
        scratch_types=[
            pltpu.VMEM((nidx, 128), jnp.int32),
            [pltpu.VMEM((_CHUNK, dim), jnp.float32) for _ in range(_NBUF)],
            [pltpu.SemaphoreType.DMA for _ in range(_NBUF)],
            [pltpu.SemaphoreType.DMA for _ in range(_NBUF)],
        ],
        compiler_params=pltpu.CompilerParams(
            use_tc_tiling_on_sc=False, needs_layout_passes=False
        ),
    )
    def gather(idx_hbm, table_hbm, out_hbm, idx_v, buf, sg, sw):
        wid = lax.axis_index("s") * _NC + lax.axis_index("c")
        i0 = pl.multiple_of(wid * nidx, 8)
        pltpu.sync_copy(idx_hbm.at[pl.ds(i0, nidx)], idx_v)

        def fire_g(k, b):
            for i in range(s_per_c):
                pltpu.async_copy(
                    table_hbm.at[idx_v.at[k * s_per_c + i]],
                    buf[b].at[pl.ds(i * 128, 128)],
                    sg[b],
                )

        def drain_g(b):
            for _ in range(s_per_c):
                pltpu.make_async_copy(
                    table_hbm.at[idx_v.at[0]], buf[b].at[pl.ds(0, 128)], sg[b]
                ).wait()

        def fire_w(k, b):
            r0 = pl.multiple_of(wid * rows_w + k * _CHUNK, 8)
            pltpu.async_copy(buf[b], out_hbm.at[pl.ds(r0, _CHUNK)], sw[b])

        def wait_w(b):
            pltpu.make_async_copy(
                buf[b], out_hbm.at[pl.ds(0, _CHUNK)], sw[b]
            ).wait()

        for b in range(_NBUF - 1):
            fire_g(b, b)

        @pl.loop(0, n // _NBUF)
        def _m(m):
            for s in range(_NBUF):
                k = m * _NBUF + s
                drain_g(s)
                fire_w(k, s)
                bn = (s + _NBUF - 1) % _NBUF

                @pl.when(jnp.logical_and(k >= 1, k < n - (_NBUF - 1)))
                def _wait():
                    wait_w(bn)

                @pl.when(k < n - (_NBUF - 1))
                def _fire():
                    fire_g(k + _NBUF - 1, bn)

        for b in range(_NBUF):
            wait_w(b)

    return gather


def kernel(indices, weight):
    batch, hist = indices.shape
    dim = weight.shape[1]
    nrows = batch * hist
    out = _make_gather(nrows, dim)(indices.reshape(-1, 128), weight)
    return out.reshape(batch, hist, dim)
